# baseline (speedup 1.0000x reference)
import math

import jax
import jax.numpy as jnp
from jax.experimental import pallas as pl

_D = 128
_NSEEDS = 500
_NPAD = 512
_K = 4
_BBLK = 256
_NEG = -1e30


def _body(x_ref, seeds_ref, wq_ref, field_ref, attn_ref):
    x = x_ref[...]
    seeds = seeds_ref[...]
    wq = wq_ref[...]
    q = jax.lax.dot_general(x, wq, (((1,), (1,)), ((), ())),
                            preferred_element_type=jnp.float32)
    s = jax.lax.dot_general(q, seeds, (((1,), (1,)), ((), ())),
                            preferred_element_type=jnp.float32)
    s = s * (1.0 / math.sqrt(_D))
    cols = jax.lax.broadcasted_iota(jnp.int32, (_BBLK, _NPAD), 1)
    s = jnp.where(cols < _NSEEDS, s, _NEG)

    v1 = jnp.max(s, axis=1, keepdims=True)
    s1 = jnp.where(s >= v1, _NEG, s)
    v2 = jnp.max(s1, axis=1, keepdims=True)
    s2 = jnp.where(s1 >= v2, _NEG, s1)
    v3 = jnp.max(s2, axis=1, keepdims=True)
    s3 = jnp.where(s2 >= v3, _NEG, s2)
    v4 = jnp.max(s3, axis=1, keepdims=True)

    e2 = jnp.exp(v2 - v1)
    e3 = jnp.exp(v3 - v1)
    e4 = jnp.exp(v4 - v1)
    rz = 1.0 / (1.0 + e2 + e3 + e4)
    attn_ref[...] = jnp.concatenate([jnp.ones_like(v1), e2, e3, e4],
                                    axis=1) * rz

    w = jnp.exp(jnp.where(s >= v4, s - v1, -100.0))
    f = jax.lax.dot_general(w, seeds, (((1,), (0,)), ((), ())),
                            preferred_element_type=jnp.float32)
    field_ref[...] = f * rz


def kernel(x, seeds, Wq):
    batch = x.shape[0]
    seeds_p = jnp.zeros((_NPAD, _D), jnp.float32).at[:_NSEEDS].set(seeds)
    grid = (batch // _BBLK,)
    field, attn = pl.pallas_call(
        _body,
        grid=grid,
        in_specs=[
            pl.BlockSpec((_BBLK, _D), lambda i: (i, 0)),
            pl.BlockSpec((_NPAD, _D), lambda i: (0, 0)),
            pl.BlockSpec((_D, _D), lambda i: (0, 0)),
        ],
        out_specs=[
            pl.BlockSpec((_BBLK, _D), lambda i: (i, 0)),
            pl.BlockSpec((_BBLK, _K), lambda i: (i, 0)),
        ],
        out_shape=[
            jax.ShapeDtypeStruct((batch, _D), jnp.float32),
            jax.ShapeDtypeStruct((batch, _K), jnp.float32),
        ],
    )(x, seeds_p, Wq)
    return (field, attn)


# speedup vs baseline: 1.6000x; 1.6000x over previous
import math

import jax
import jax.numpy as jnp
from jax.experimental import pallas as pl

_D = 128
_NSEEDS = 500
_NPAD = 512
_K = 4
_BBLK = 1024
_NEG = -1e30


def _body(x_ref, seeds_ref, wq_ref, field_ref, attn_ref):
    x = x_ref[...]
    seeds = seeds_ref[...]
    wq = wq_ref[...]
    q = jax.lax.dot_general(x, wq, (((1,), (1,)), ((), ())),
                            preferred_element_type=jnp.float32)
    s = jax.lax.dot_general(q, seeds, (((1,), (1,)), ((), ())),
                            preferred_element_type=jnp.float32)
    s = s * (1.0 / math.sqrt(_D))
    cols = jax.lax.broadcasted_iota(jnp.int32, (_BBLK, _NPAD), 1)
    s = jnp.where(cols < _NSEEDS, s, _NEG)

    v1 = jnp.max(s, axis=1, keepdims=True)
    s1 = jnp.where(s >= v1, _NEG, s)
    v2 = jnp.max(s1, axis=1, keepdims=True)
    s2 = jnp.where(s1 >= v2, _NEG, s1)
    v3 = jnp.max(s2, axis=1, keepdims=True)
    s3 = jnp.where(s2 >= v3, _NEG, s2)
    v4 = jnp.max(s3, axis=1, keepdims=True)

    e2 = jnp.exp(v2 - v1)
    e3 = jnp.exp(v3 - v1)
    e4 = jnp.exp(v4 - v1)
    rz = 1.0 / (1.0 + e2 + e3 + e4)
    attn_ref[...] = jnp.concatenate([jnp.ones_like(v1), e2, e3, e4],
                                    axis=1) * rz

    w = jnp.exp(jnp.where(s >= v4, s - v1, -100.0))
    f = jax.lax.dot_general(w, seeds, (((1,), (0,)), ((), ())),
                            preferred_element_type=jnp.float32)
    field_ref[...] = f * rz


def kernel(x, seeds, Wq):
    batch = x.shape[0]
    seeds_p = jnp.zeros((_NPAD, _D), jnp.float32).at[:_NSEEDS].set(seeds)
    grid = (batch // _BBLK,)
    field, attn = pl.pallas_call(
        _body,
        grid=grid,
        in_specs=[
            pl.BlockSpec((_BBLK, _D), lambda i: (i, 0)),
            pl.BlockSpec((_NPAD, _D), lambda i: (0, 0)),
            pl.BlockSpec((_D, _D), lambda i: (0, 0)),
        ],
        out_specs=[
            pl.BlockSpec((_BBLK, _D), lambda i: (i, 0)),
            pl.BlockSpec((_BBLK, _K), lambda i: (i, 0)),
        ],
        out_shape=[
            jax.ShapeDtypeStruct((batch, _D), jnp.float32),
            jax.ShapeDtypeStruct((batch, _K), jnp.float32),
        ],
    )(x, seeds_p, Wq)
    return (field, attn)


# speedup vs baseline: 1.6728x; 1.0455x over previous
import math

import jax
import jax.numpy as jnp
from jax.experimental import pallas as pl

_D = 128
_NSEEDS = 500
_NPAD = 512
_K = 4
_BBLK = 2048
_NEG = -1e30


def _body(x_ref, seeds_ref, wq_ref, field_ref, attn_ref):
    x = x_ref[...]
    seeds = seeds_ref[...]
    wq = wq_ref[...]
    q = jax.lax.dot_general(x, wq, (((1,), (1,)), ((), ())),
                            preferred_element_type=jnp.float32)
    s = jax.lax.dot_general(q, seeds, (((1,), (1,)), ((), ())),
                            preferred_element_type=jnp.float32)
    s = s * (1.0 / math.sqrt(_D))
    cols = jax.lax.broadcasted_iota(jnp.int32, (_BBLK, _NPAD), 1)
    s = jnp.where(cols < _NSEEDS, s, _NEG)

    v1 = jnp.max(s, axis=1, keepdims=True)
    s1 = jnp.where(s >= v1, _NEG, s)
    v2 = jnp.max(s1, axis=1, keepdims=True)
    s2 = jnp.where(s1 >= v2, _NEG, s1)
    v3 = jnp.max(s2, axis=1, keepdims=True)
    s3 = jnp.where(s2 >= v3, _NEG, s2)
    v4 = jnp.max(s3, axis=1, keepdims=True)

    e2 = jnp.exp(v2 - v1)
    e3 = jnp.exp(v3 - v1)
    e4 = jnp.exp(v4 - v1)
    rz = 1.0 / (1.0 + e2 + e3 + e4)
    attn_ref[...] = jnp.concatenate([jnp.ones_like(v1), e2, e3, e4],
                                    axis=1) * rz

    w = jnp.exp(jnp.where(s >= v4, s - v1, -100.0))
    f = jax.lax.dot_general(w, seeds, (((1,), (0,)), ((), ())),
                            preferred_element_type=jnp.float32)
    field_ref[...] = f * rz


def kernel(x, seeds, Wq):
    batch = x.shape[0]
    seeds_p = jnp.zeros((_NPAD, _D), jnp.float32).at[:_NSEEDS].set(seeds)
    grid = (batch // _BBLK,)
    field, attn = pl.pallas_call(
        _body,
        grid=grid,
        in_specs=[
            pl.BlockSpec((_BBLK, _D), lambda i: (i, 0)),
            pl.BlockSpec((_NPAD, _D), lambda i: (0, 0)),
            pl.BlockSpec((_D, _D), lambda i: (0, 0)),
        ],
        out_specs=[
            pl.BlockSpec((_BBLK, _D), lambda i: (i, 0)),
            pl.BlockSpec((_BBLK, _K), lambda i: (i, 0)),
        ],
        out_shape=[
            jax.ShapeDtypeStruct((batch, _D), jnp.float32),
            jax.ShapeDtypeStruct((batch, _K), jnp.float32),
        ],
    )(x, seeds_p, Wq)
    return (field, attn)
